# baseline (speedup 1.0000x reference)
import jax
import jax.numpy as jnp
from jax.experimental import pallas as pl

N_NODES = 2048
N_EDGES = 65536
N_GRAPHS = 128
NODES_PER_GRAPH = 16
EMB = 64
HID = 64
N_LAYERS = 8
K = 16
HEADS = 4
ATT_DIM = N_LAYERS * HID


def _segment_softmax(scores, seg, num):
    m = jax.ops.segment_max(scores, seg, num_segments=num)
    m = jnp.where(jnp.isfinite(m), m, 0.0)
    ex = jnp.exp(scores - m[seg])
    s = jax.ops.segment_sum(ex, seg, num_segments=num)
    return ex / (s[seg] + 1e-16)


def _egat_layer(h, e, src, dst, Wns, Wni, Wfij, Wnj, attn, bias):
    f_ni = h @ Wni.T
    f_nj = h @ Wnj.T
    f_fij = e @ Wfij.T
    f_out = f_ni[src] + f_nj[dst] + f_fij + bias
    ee = jax.nn.leaky_relu(f_out, 0.01)
    a = (ee * attn).sum(-1)
    a = _segment_softmax(a, dst, N_NODES)
    ft = h @ Wns.T
    out = jax.ops.segment_sum(ft[src] * a[:, None], dst, num_segments=N_NODES)
    return out, ee


def _full_qk_attention(x, Wqk, Wv, Wout, bout):
    b, t, dim = x.shape
    qk = x @ Wqk.T
    v = x @ Wv.T
    dh = dim // HEADS

    def split(z):
        return z.reshape(b, t, HEADS, dh).transpose(0, 2, 1, 3).reshape(b * HEADS, t, dh)

    qk = split(qk)
    v = split(v)
    q = qk
    k = qk / jnp.maximum(jnp.linalg.norm(qk, axis=-1, keepdims=True), 1e-12)
    dot = jnp.einsum('bie,bje->bij', q, k) * (dh ** -0.5)
    i = jnp.arange(t)
    dot = dot.at[:, i, i].set(-5e4)
    causal = jnp.tril(jnp.ones((t, t), dtype=bool))
    dot = jnp.where(causal[None], dot, jnp.finfo(dot.dtype).min)
    attn = jax.nn.softmax(dot, axis=-1)
    out = jnp.einsum('bij,bje->bie', attn, v)
    out = out.reshape(b, HEADS, t, dh).transpose(0, 2, 1, 3).reshape(b, t, dim)
    return out @ Wout.T + bout


def _sortpool(h):
    feat = jnp.sort(h, axis=-1)
    g = feat.reshape(N_GRAPHS, NODES_PER_GRAPH, HID)
    order = jnp.argsort(-g[:, :, -1], axis=1)
    g = jnp.take_along_axis(g, order[:, :, None], axis=1)
    return g[:, :K, :].reshape(N_GRAPHS, K * HID)


def _gat_layer(h, src, dst, W, al, ar, bias, n):
    ft = h @ W.T
    el = (ft * al).sum(-1)
    er = (ft * ar).sum(-1)
    sc = jax.nn.leaky_relu(el[src] + er[dst], 0.2)
    a = _segment_softmax(sc, dst, n)
    out = jax.ops.segment_sum(ft[src] * a[:, None], dst, num_segments=n)
    return out + bias


def _head_kernel(h_ref, wlin_ref, blin_ref, wlin1_ref, blin1_ref,
                 wcls_ref, bcls_ref, out_ref):
    h = h_ref[...]
    h = jnp.maximum(jnp.dot(h, wlin_ref[...].T,
                            preferred_element_type=jnp.float32) + blin_ref[...], 0.0)
    h = jnp.maximum(jnp.dot(h, wlin1_ref[...].T,
                            preferred_element_type=jnp.float32) + blin1_ref[...], 0.0)
    out_ref[...] = jnp.dot(h, wcls_ref[...].T,
                           preferred_element_type=jnp.float32) + bcls_ref[...]


def _head(h, Wlin, blin, Wlin1, blin1, Wcls, bcls):
    return pl.pallas_call(
        _head_kernel,
        out_shape=jax.ShapeDtypeStruct((N_GRAPHS, 2), jnp.float32),
    )(h, Wlin, blin[None], Wlin1, blin1[None], Wcls, bcls[None])


def kernel(token_emb, e_token_emb, egat_Wns, egat_Wni, egat_Wfij, egat_Wnj, egat_attn, egat_bias, Wqk, Wv, Wout, bout, Wli1, bli1, Wli2, bli2, Wlf1, blf1, Wg, gal, gar, gbias, Wlin, blin, Wlin1, blin1, Wcls, bcls, h_tok, e_tok, edge_index, fg_edge_index):
    src, dst = edge_index[0], edge_index[1]
    h = jax.nn.relu(token_emb[h_tok])
    e = e_token_emb[e_tok]
    hs = []
    for i in range(N_LAYERS):
        h, e = _egat_layer(h, e, src, dst, egat_Wns[i], egat_Wni[i],
                           egat_Wfij[i], egat_Wnj[i], egat_attn[i], egat_bias[i])
        h = jax.nn.relu(h)
        hs.append(h)
    hs = jnp.concatenate(hs, axis=-1)
    a = _full_qk_attention(hs[None], Wqk, Wv, Wout, bout)[0]
    a = jax.nn.gelu(a @ Wli1.T + bli1, approximate=False)
    a = jax.nn.gelu(a @ Wli2.T + bli2, approximate=False)
    h = jax.nn.relu(a @ Wlf1.T + blf1)
    h = _sortpool(h)
    h = jax.nn.relu(_gat_layer(h, fg_edge_index[0], fg_edge_index[1],
                               Wg, gal, gar, gbias, N_GRAPHS))
    h = _head(h, Wlin, blin, Wlin1, blin1, Wcls, bcls)
    return h.reshape(-1, 2)


# speedup vs baseline: 9.5194x; 9.5194x over previous
import functools

import jax
import jax.numpy as jnp
from jax import lax
from jax.experimental import pallas as pl
from jax.experimental.pallas import tpu as pltpu
from jax.experimental.pallas import tpu_sc as plsc

N_NODES = 2048
N_EDGES = 65536
N_GRAPHS = 128
NPG = 16
HID = 64
N_LAYERS = 8
HEADS = 4
ATT_DIM = N_LAYERS * HID
DH = ATT_DIM // HEADS

F32 = jnp.float32
NEG_BIG = float(jnp.finfo(jnp.float32).min)

N_TILES = 32
EPT = N_EDGES // N_TILES
CHUNK = 128
N_CHUNKS = EPT // CHUNK
ACC_W = 128

@functools.cache
def _sc_mesh():
    return plsc.VectorSubcoreMesh(core_axis_name="c", subcore_axis_name="s")


def _lane_perm(v, idx):
    return lax.gather(
        v, idx[:, None],
        lax.GatherDimensionNumbers(offset_dims=(), collapsed_slice_dims=(0,),
                                   start_index_map=(0,)),
        slice_sizes=(1,), mode=lax.GatherScatterMode.PROMISE_IN_BOUNDS)


def _emb_body(htok, etok, htab, etab, h0_out, e0_out,
              hib, hrows, ib, rows, sem):
    cid = lax.axis_index("c")
    sid = lax.axis_index("s")
    wid = sid * 2 + cid
    hbase = wid * (N_NODES // N_TILES)
    pltpu.sync_copy(htok.at[pl.ds(hbase, N_NODES // N_TILES)], hib)
    pltpu.async_copy(htab.at[hib], hrows, sem).wait()
    pltpu.sync_copy(hrows, h0_out.at[pl.ds(hbase, N_NODES // N_TILES)])
    for c in range(N_CHUNKS):
        base = wid * EPT + c * CHUNK
        pltpu.sync_copy(etok.at[pl.ds(base, CHUNK)], ib)
        pltpu.async_copy(etab.at[ib], rows, sem).wait()
        pltpu.sync_copy(rows, e0_out.at[pl.ds(base, CHUNK)])


@functools.cache
def _emb_call():
    return pl.kernel(
        _emb_body,
        out_type=(
            jax.ShapeDtypeStruct((N_NODES, ACC_W), F32),
            jax.ShapeDtypeStruct((N_EDGES, ACC_W), F32),
        ),
        mesh=_sc_mesh(),
        scratch_types=[
            pltpu.VMEM((N_NODES // N_TILES,), jnp.int32),
            pltpu.VMEM((N_NODES // N_TILES, ACC_W), F32),
            pltpu.VMEM((CHUNK,), jnp.int32),
            pltpu.VMEM((CHUNK, ACC_W), F32),
            pltpu.SemaphoreType.DMA,
        ],
    )


def _edge_body(fsrc, fdst, ffij, src, dst, attn, zeros, ee_out, acc_out,
               attn_v, sbuf, dbuf, A, B, Fb, EE, R, acc_sh, sem):
    cid = lax.axis_index("c")
    sid = lax.axis_index("s")
    wid = sid * 2 + cid
    pltpu.sync_copy(zeros, acc_sh.at[pl.ds(sid * CHUNK, CHUNK)])
    pltpu.sync_copy(attn, attn_v)

    def zrow(r, carry):
        for k in range(HID // 16, ACC_W // 16):
            R[r, pl.ds(k * 16, 16)] = jnp.zeros((16,), F32)
        return carry

    lax.fori_loop(0, CHUNK, zrow, 0)
    plsc.subcore_barrier()
    lane = lax.broadcasted_iota(jnp.int32, (16,), 0)
    for c in range(N_CHUNKS):
        base = wid * EPT + c * CHUNK
        pltpu.sync_copy(src.at[pl.ds(base, CHUNK)], sbuf)
        pltpu.sync_copy(dst.at[pl.ds(base, CHUNK)], dbuf)
        cp1 = pltpu.async_copy(fsrc.at[sbuf], A, sem)
        cp2 = pltpu.async_copy(fdst.at[dbuf], B, sem)
        cp3 = pltpu.async_copy(ffij.at[pl.ds(base, CHUNK)], Fb, sem)
        cp1.wait()
        cp2.wait()
        cp3.wait()

        def edge_step(r, carry):
            sacc = jnp.zeros((16,), F32)
            for k in range(HID // 16):
                x = A[r, pl.ds(k * 16, 16)] + B[r, pl.ds(k * 16, 16)] \
                    + Fb[r, pl.ds(k * 16, 16)]
                eev = jnp.maximum(x, x * 0.01)
                EE[r, pl.ds(k * 16, 16)] = eev
                sacc = sacc + eev * attn_v[pl.ds(k * 16, 16)]
            for d in (1, 2, 4, 8):
                sacc = sacc + _lane_perm(sacc, jnp.bitwise_xor(lane, d))
            exv = jnp.exp(sacc)
            for k in range(HID // 16):
                R[r, pl.ds(k * 16, 16)] = A[r, pl.ds(HID + k * 16, 16)] * exv
            R[r, pl.ds(HID, 16)] = jnp.where(lane == 0, exv, 0.0)
            return carry

        lax.fori_loop(0, CHUNK, edge_step, 0)
        pltpu.sync_copy(R, acc_sh.at[dbuf], add=True)
        pltpu.sync_copy(EE, ee_out.at[pl.ds(base, CHUNK)])
    plsc.subcore_barrier()
    pltpu.sync_copy(acc_sh.at[pl.ds(sid * CHUNK, CHUNK)],
                    acc_out.at[cid, pl.ds(sid * CHUNK, CHUNK)])


@functools.cache
def _edge_call():
    return pl.kernel(
        _edge_body,
        out_type=(
            jax.ShapeDtypeStruct((N_EDGES, HID), F32),
            jax.ShapeDtypeStruct((2, N_NODES, ACC_W), F32),
        ),
        mesh=_sc_mesh(),
        scratch_types=[
            pltpu.VMEM((HID,), F32),
            pltpu.VMEM((CHUNK,), jnp.int32),
            pltpu.VMEM((CHUNK,), jnp.int32),
            pltpu.VMEM((CHUNK, ACC_W), F32),
            pltpu.VMEM((CHUNK, ACC_W), F32),
            pltpu.VMEM((CHUNK, HID), F32),
            pltpu.VMEM((CHUNK, HID), F32),
            pltpu.VMEM((CHUNK, ACC_W), F32),
            pltpu.VMEM_SHARED((N_NODES, ACC_W), F32),
            pltpu.SemaphoreType.DMA,
        ],
    )


def _dotT(x, w):
    return lax.dot_general(x, w, (((1,), (1,)), ((), ())),
                           preferred_element_type=F32)


def _node_prep0_body(h0_ref, wni_ref, wnj_ref, wns_ref,
                     h_ref, fsrc_ref, fdst_ref):
    h = jnp.maximum(h0_ref[:, :HID], 0.0)
    h_ref[...] = h
    fni = _dotT(h, wni_ref[...])
    fnj = _dotT(h, wnj_ref[...])
    ft = _dotT(h, wns_ref[...])
    fsrc_ref[...] = jnp.concatenate([fni, ft], axis=1)
    fdst_ref[...] = jnp.concatenate([fnj, jnp.zeros_like(fnj)], axis=1)


def _node_prep_body(acc0_ref, acc1_ref, wni_ref, wnj_ref, wns_ref,
                    h_ref, fsrc_ref, fdst_ref):
    acc = acc0_ref[...] + acc1_ref[...]
    num = acc[:, :HID]
    den = acc[:, HID:HID + 1]
    h = jnp.maximum(num / (den + 1e-16), 0.0)
    h_ref[...] = h
    fni = _dotT(h, wni_ref[...])
    fnj = _dotT(h, wnj_ref[...])
    ft = _dotT(h, wns_ref[...])
    fsrc_ref[...] = jnp.concatenate([fni, ft], axis=1)
    fdst_ref[...] = jnp.concatenate([fnj, jnp.zeros_like(fnj)], axis=1)


def _finalize_body(acc0_ref, acc1_ref, h_ref):
    acc = acc0_ref[...] + acc1_ref[...]
    h_ref[...] = jnp.maximum(acc[:, :HID] / (acc[:, HID:HID + 1] + 1e-16), 0.0)


def _edge_mm_body(e_ref, w_ref, out_ref):
    out_ref[...] = _dotT(e_ref[:, :HID], w_ref[...])


def _attn_proj_body(x_ref, wqk_ref, wv_ref, qk_ref, kn_ref, v_ref):
    x = x_ref[...]
    qk = _dotT(x, wqk_ref[...])
    qk_ref[...] = qk
    v_ref[...] = _dotT(x, wv_ref[...])
    for hd in range(HEADS):
        s = qk[:, hd * DH:(hd + 1) * DH]
        nrm = jnp.sqrt(jnp.sum(s * s, axis=1, keepdims=True))
        kn_ref[:, hd * DH:(hd + 1) * DH] = s / jnp.maximum(nrm, 1e-12)


def _attention_body(qk_ref, kn_ref, v_ref, out_ref):
    qb = pl.program_id(1)
    q = qk_ref[...]
    scores = lax.dot_general(q, kn_ref[...], (((1,), (1,)), ((), ())),
                             preferred_element_type=F32) * (DH ** -0.5)
    row = lax.broadcasted_iota(jnp.int32, scores.shape, 0) + qb * scores.shape[0]
    col = lax.broadcasted_iota(jnp.int32, scores.shape, 1)
    scores = jnp.where(col == row, -5e4, scores)
    scores = jnp.where(col <= row, scores, NEG_BIG)
    m = jnp.max(scores, axis=1, keepdims=True)
    p = jnp.exp(scores - m)
    out = jnp.dot(p, v_ref[...], preferred_element_type=F32)
    out_ref[...] = out / jnp.sum(p, axis=1, keepdims=True)


def _erf(x):
    a1, a2, a3 = 0.254829592, -0.284496736, 1.421413741
    a4, a5, p = -1.453152027, 1.061405429, 0.3275911
    s = jnp.sign(x)
    ax = jnp.abs(x)
    t = 1.0 / (1.0 + p * ax)
    y = 1.0 - (((((a5 * t + a4) * t) + a3) * t + a2) * t + a1) * t * jnp.exp(-ax * ax)
    return s * y


def _gelu(x):
    return 0.5 * x * (1.0 + _erf(x * (2.0 ** -0.5)))


def _post_attn_body(x_ref, wout_ref, bout_ref, wli1_ref, bli1_ref,
                    wli2_ref, bli2_ref, wlf1_ref, blf1_ref, out_ref):
    a = _dotT(x_ref[...], wout_ref[...]) + bout_ref[...]
    a = _gelu(_dotT(a, wli1_ref[...]) + bli1_ref[...])
    a = _gelu(_dotT(a, wli2_ref[...]) + bli2_ref[...])
    out_ref[...] = jnp.maximum(_dotT(a, wlf1_ref[...]) + blf1_ref[...], 0.0)


def _sortpool_body(h_ref, out_ref):
    feat = h_ref[...]
    lane = lax.broadcasted_iota(jnp.int32, (N_NODES, HID), 1)
    r64 = lax.broadcasted_iota(jnp.int32, (HID, HID), 0)
    c64 = lax.broadcasted_iota(jnp.int32, (HID, HID), 1)
    k = 2
    while k <= HID:
        j = k // 2
        while j >= 1:
            pj = (jnp.bitwise_xor(r64, j) == c64).astype(F32)
            xp = jnp.dot(feat, pj, preferred_element_type=F32)
            bitj0 = (lane & j) == 0
            dirup = (lane & k) == 0
            cond = bitj0 == dirup
            feat = jnp.where(cond, jnp.minimum(feat, xp),
                             jnp.maximum(feat, xp))
            j //= 2
        k *= 2
    kcol = feat[:, HID - 1:HID]
    sub = lax.broadcasted_iota(jnp.int32, (N_NODES, NPG), 0)
    ln16 = lax.broadcasted_iota(jnp.int32, (N_NODES, NPG), 1)
    m16 = jnp.where((sub % NPG) == ln16, kcol, 0.0)
    gsel_r = lax.broadcasted_iota(jnp.int32, (N_GRAPHS, N_NODES), 0)
    gsel_c = lax.broadcasted_iota(jnp.int32, (N_GRAPHS, N_NODES), 1)
    gsel = ((gsel_c // NPG) == gsel_r).astype(F32)
    kk = jnp.dot(gsel, m16, preferred_element_type=F32)
    l16 = lax.broadcasted_iota(jnp.int32, (N_GRAPHS, NPG), 1)
    rank = jnp.zeros((N_GRAPHS, NPG), jnp.int32)
    for j in range(NPG):
        kj = kk[:, j:j + 1]
        gt = (kj > kk).astype(jnp.int32)
        eq = jnp.logical_and(kj == kk, j < l16).astype(jnp.int32)
        rank = rank + gt + eq
    q_r = lax.broadcasted_iota(jnp.int32, (NPG, N_NODES), 0)
    q_c = lax.broadcasted_iota(jnp.int32, (NPG, N_NODES), 1)
    qsel = ((q_c % NPG) == q_r).astype(F32)
    rexp = jnp.dot(rank.astype(F32), qsel,
                   preferred_element_type=F32)
    gmask = ((gsel_c // NPG) == gsel_r).astype(F32)
    ones_row = jnp.ones((1, N_GRAPHS), F32)
    rank_row = jnp.dot(ones_row, rexp * gmask,
                       preferred_element_type=F32)
    blk = 256
    for b in range(N_NODES // blk):
        pr = lax.broadcasted_iota(jnp.int32, (blk, N_NODES), 0) + b * blk
        pc = lax.broadcasted_iota(jnp.int32, (blk, N_NODES), 1)
        same_g = (pr // NPG) == (pc // NPG)
        sel = jnp.logical_and(same_g, rank_row == (pr % NPG).astype(F32))
        out_ref[pl.ds(b * blk, blk), :] = jnp.dot(
            sel.astype(F32), feat, preferred_element_type=F32)


def _gat_head_body(hsp_ref, hspT_ref, srow_ref, drow_ref, scol_ref, dcol_ref,
                   wg_ref, gal_ref, galr_ref, gar_ref, garr_ref, gb_ref,
                   wlin_ref, blin_ref, wlin1_ref, blin1_ref,
                   wcls_ref, bcls_ref, out_ref):
    hsp = hsp_ref[...]
    hspT = hspT_ref[...]
    wg = wg_ref[...]
    ft = _dotT(hsp, wg)
    ftT = jnp.dot(wg, hspT, preferred_element_type=F32)
    el_c = jnp.dot(ft, gal_ref[...], preferred_element_type=F32)
    er_c = jnp.dot(ft, gar_ref[...], preferred_element_type=F32)
    el_r = jnp.dot(galr_ref[...], ftT, preferred_element_type=F32)
    er_r = jnp.dot(garr_ref[...], ftT, preferred_element_type=F32)
    srow = srow_ref[...]
    drow = drow_ref[...]
    scol = scol_ref[...]
    dcol = dcol_ref[...]
    n_r = lax.broadcasted_iota(jnp.int32, (N_GRAPHS, 8 * N_GRAPHS), 0)
    n_c = lax.broadcasted_iota(jnp.int32, (8 * N_GRAPHS, N_GRAPHS), 1)
    o_src = (srow == n_r).astype(F32)
    o_dst = (drow == n_r).astype(F32)
    o_srcT = (scol == n_c).astype(F32)
    o_dstT = (dcol == n_c).astype(F32)
    sc_r = jnp.dot(el_r, o_src, preferred_element_type=F32) \
        + jnp.dot(er_r, o_dst, preferred_element_type=F32)
    sc_r = jnp.maximum(sc_r, sc_r * 0.2)
    sc_c = jnp.dot(o_srcT, el_c, preferred_element_type=F32) \
        + jnp.dot(o_dstT, er_c, preferred_element_type=F32)
    sc_c = jnp.maximum(sc_c, sc_c * 0.2)
    mmat = jnp.where(o_dst > 0.5, sc_r, NEG_BIG)
    m_c = jnp.max(mmat, axis=1, keepdims=True)
    mmatT = jnp.where(o_dstT > 0.5, sc_c, NEG_BIG)
    m_r = jnp.max(mmatT, axis=0, keepdims=True)
    m_c = jnp.where(m_c <= NEG_BIG, 0.0, m_c)
    m_r = jnp.where(m_r <= NEG_BIG, 0.0, m_r)
    ex_c = jnp.exp(sc_c - jnp.dot(o_dstT, m_c, preferred_element_type=F32))
    s_c = jnp.dot(o_dst, ex_c, preferred_element_type=F32)
    den_c = jnp.dot(o_dstT, s_c, preferred_element_type=F32) + 1e-16
    a_c = ex_c / den_c
    ft_src = jnp.dot(o_srcT, ft, preferred_element_type=F32)
    out = jnp.dot(o_dst, ft_src * a_c, preferred_element_type=F32)
    h = jnp.maximum(out + gb_ref[...], 0.0)
    h = jnp.maximum(_dotT(h, wlin_ref[...]) + blin_ref[...], 0.0)
    h = jnp.maximum(_dotT(h, wlin1_ref[...]) + blin1_ref[...], 0.0)
    out_ref[...] = _dotT(h, wcls_ref[...]) + bcls_ref[...]


def _tc_call(body, out_shape, **kw):
    return pl.pallas_call(body, out_shape=out_shape, **kw)


def kernel(token_emb, e_token_emb, egat_Wns, egat_Wni, egat_Wfij, egat_Wnj,
           egat_attn, egat_bias, Wqk, Wv, Wout, bout, Wli1, bli1, Wli2, bli2,
           Wlf1, blf1, Wg, gal, gar, gbias, Wlin, blin, Wlin1, blin1,
           Wcls, bcls, h_tok, e_tok, edge_index, fg_edge_index):
    f32 = jnp.float32
    src = edge_index[0]
    dst = edge_index[1]
    zeros_init = jnp.zeros((CHUNK, ACC_W), f32)
    htab = jnp.pad(token_emb.astype(f32), ((0, 0), (0, ACC_W - HID)))
    etab = jnp.pad(e_token_emb.astype(f32), ((0, 0), (0, ACC_W - HID)))

    h0_raw, e = _emb_call()(h_tok, e_tok, htab, etab)

    node_shapes = (
        jax.ShapeDtypeStruct((N_NODES, HID), f32),
        jax.ShapeDtypeStruct((N_NODES, 2 * HID), f32),
        jax.ShapeDtypeStruct((N_NODES, 2 * HID), f32),
    )

    hs_list = []
    acc = None
    for l in range(N_LAYERS):
        if l == 0:
            h, fsrc, fdst = _tc_call(_node_prep0_body, node_shapes)(
                h0_raw, egat_Wni[l], egat_Wnj[l], egat_Wns[l])
        else:
            h, fsrc, fdst = _tc_call(_node_prep_body, node_shapes)(
                acc[0], acc[1], egat_Wni[l], egat_Wnj[l], egat_Wns[l])
            hs_list.append(h)
        e_w = e.shape[1]
        ffij = _tc_call(
            _edge_mm_body,
            jax.ShapeDtypeStruct((N_EDGES, HID), f32),
            grid=(8,),
            in_specs=[
                pl.BlockSpec((N_EDGES // 8, e_w), lambda i: (i, 0)),
                pl.BlockSpec((HID, HID), lambda i: (0, 0)),
            ],
            out_specs=pl.BlockSpec((N_EDGES // 8, HID), lambda i: (i, 0)),
        )(e, egat_Wfij[l])
        e, acc = _edge_call()(fsrc, fdst, ffij, src, dst,
                              egat_attn[l], zeros_init)

    h8 = _tc_call(_finalize_body, jax.ShapeDtypeStruct((N_NODES, HID), f32))(
        acc[0], acc[1])
    hs_list.append(h8)
    hs = jnp.concatenate(hs_list, axis=-1)

    qk, kn, v = _tc_call(
        _attn_proj_body,
        (jax.ShapeDtypeStruct((N_NODES, ATT_DIM), f32),) * 3,
        grid=(8,),
        in_specs=[
            pl.BlockSpec((N_NODES // 8, ATT_DIM), lambda i: (i, 0)),
            pl.BlockSpec((ATT_DIM, ATT_DIM), lambda i: (0, 0)),
            pl.BlockSpec((ATT_DIM, ATT_DIM), lambda i: (0, 0)),
        ],
        out_specs=[pl.BlockSpec((N_NODES // 8, ATT_DIM), lambda i: (i, 0))] * 3,
    )(hs, Wqk, Wv)

    att = _tc_call(
        _attention_body,
        jax.ShapeDtypeStruct((N_NODES, ATT_DIM), f32),
        grid=(HEADS, 8),
        in_specs=[
            pl.BlockSpec((N_NODES // 8, DH), lambda hd, q: (q, hd)),
            pl.BlockSpec((N_NODES, DH), lambda hd, q: (0, hd)),
            pl.BlockSpec((N_NODES, DH), lambda hd, q: (0, hd)),
        ],
        out_specs=pl.BlockSpec((N_NODES // 8, DH), lambda hd, q: (q, hd)),
    )(qk, kn, v)

    hp = _tc_call(
        _post_attn_body,
        jax.ShapeDtypeStruct((N_NODES, HID), f32),
        grid=(8,),
        in_specs=[
            pl.BlockSpec((N_NODES // 8, ATT_DIM), lambda i: (i, 0)),
            pl.BlockSpec((ATT_DIM, ATT_DIM), lambda i: (0, 0)),
            pl.BlockSpec((1, ATT_DIM), lambda i: (0, 0)),
            pl.BlockSpec((HID, ATT_DIM), lambda i: (0, 0)),
            pl.BlockSpec((1, HID), lambda i: (0, 0)),
            pl.BlockSpec((ATT_DIM, HID), lambda i: (0, 0)),
            pl.BlockSpec((1, ATT_DIM), lambda i: (0, 0)),
            pl.BlockSpec((HID, ATT_DIM), lambda i: (0, 0)),
            pl.BlockSpec((1, HID), lambda i: (0, 0)),
        ],
        out_specs=pl.BlockSpec((N_NODES // 8, HID), lambda i: (i, 0)),
    )(att, Wout, bout[None], Wli1, bli1[None], Wli2, bli2[None],
      Wlf1, blf1[None])

    sp = _tc_call(_sortpool_body,
                  jax.ShapeDtypeStruct((N_NODES, HID), f32))(hp)
    hsp = sp.reshape(N_GRAPHS, NPG * HID)

    logits = _tc_call(_gat_head_body,
                      jax.ShapeDtypeStruct((N_GRAPHS, 2), f32))(
        hsp, hsp.T,
        fg_edge_index[0:1, :], fg_edge_index[1:2, :],
        fg_edge_index.T[:, 0:1], fg_edge_index.T[:, 1:2],
        Wg, gal[:, None], gal[None, :], gar[:, None], gar[None, :],
        gbias[None, :],
        Wlin, blin[None, :], Wlin1, blin1[None, :], Wcls, bcls[None, :])

    return logits.reshape(-1, 2)


# speedup vs baseline: 10.7622x; 1.1305x over previous
import functools

import jax
import jax.numpy as jnp
from jax import lax
from jax.experimental import pallas as pl
from jax.experimental.pallas import tpu as pltpu
from jax.experimental.pallas import tpu_sc as plsc

N_NODES = 2048
N_EDGES = 65536
N_GRAPHS = 128
NPG = 16
HID = 64
N_LAYERS = 8
HEADS = 4
ATT_DIM = N_LAYERS * HID
DH = ATT_DIM // HEADS

F32 = jnp.float32
NEG_BIG = float(jnp.finfo(jnp.float32).min)

N_TILES = 32
EPT = N_EDGES // N_TILES
CHUNK = 128
N_CHUNKS = EPT // CHUNK
ACC_W = 128

@functools.cache
def _sc_mesh():
    return plsc.VectorSubcoreMesh(core_axis_name="c", subcore_axis_name="s")


def _lane_perm(v, idx):
    return lax.gather(
        v, idx[:, None],
        lax.GatherDimensionNumbers(offset_dims=(), collapsed_slice_dims=(0,),
                                   start_index_map=(0,)),
        slice_sizes=(1,), mode=lax.GatherScatterMode.PROMISE_IN_BOUNDS)


def _emb_body(htok, etok, htab, etab, h0_out, e0_out,
              hib, hrows, ib, rows, sem):
    cid = lax.axis_index("c")
    sid = lax.axis_index("s")
    wid = sid * 2 + cid
    hbase = wid * (N_NODES // N_TILES)
    pltpu.sync_copy(htok.at[pl.ds(hbase, N_NODES // N_TILES)], hib)
    pltpu.async_copy(htab.at[hib], hrows, sem).wait()
    pltpu.sync_copy(hrows, h0_out.at[pl.ds(hbase, N_NODES // N_TILES)])
    for c in range(N_CHUNKS):
        base = wid * EPT + c * CHUNK
        pltpu.sync_copy(etok.at[pl.ds(base, CHUNK)], ib)
        pltpu.async_copy(etab.at[ib], rows, sem).wait()
        pltpu.sync_copy(rows, e0_out.at[pl.ds(base, CHUNK)])


@functools.cache
def _emb_call():
    return pl.kernel(
        _emb_body,
        out_type=(
            jax.ShapeDtypeStruct((N_NODES, ACC_W), F32),
            jax.ShapeDtypeStruct((N_EDGES, ACC_W), F32),
        ),
        mesh=_sc_mesh(),
        scratch_types=[
            pltpu.VMEM((N_NODES // N_TILES,), jnp.int32),
            pltpu.VMEM((N_NODES // N_TILES, ACC_W), F32),
            pltpu.VMEM((CHUNK,), jnp.int32),
            pltpu.VMEM((CHUNK, ACC_W), F32),
            pltpu.SemaphoreType.DMA,
        ],
    )


def _edge_body(fsrc, fdst, ffij, src, dst, attn, zeros, ee_out, acc_out,
               attn_v, sbuf, dbuf, A, B, Fb, R, acc_sh, gsem):
    cid = lax.axis_index("c")
    sid = lax.axis_index("s")
    wid = sid * 2 + cid
    pltpu.sync_copy(zeros, acc_sh.at[pl.ds(sid * CHUNK, CHUNK)])
    pltpu.sync_copy(attn, attn_v)

    def zrow(r, carry):
        for k in range(HID // 16, ACC_W // 16):
            R[r, pl.ds(k * 16, 16)] = jnp.zeros((16,), F32)
        return carry

    lax.fori_loop(0, CHUNK, zrow, 0)
    plsc.subcore_barrier()
    lane = lax.broadcasted_iota(jnp.int32, (16,), 0)
    xor_idx = [jnp.bitwise_xor(lane, d) for d in (1, 2, 4, 8)]

    def start_gathers(c, b):
        base = wid * EPT + c * CHUNK
        pltpu.sync_copy(src.at[pl.ds(base, CHUNK)], sbuf[b])
        pltpu.sync_copy(dst.at[pl.ds(base, CHUNK)], dbuf[b])
        return (pltpu.async_copy(fsrc.at[sbuf[b]], A[b], gsem[b]),
                pltpu.async_copy(fdst.at[dbuf[b]], B[b], gsem[b]))

    pend = start_gathers(0, 0)
    for c in range(N_CHUNKS):
        b = c % 2
        base_c = wid * EPT + c * CHUNK
        pltpu.sync_copy(ffij.at[pl.ds(base_c, CHUNK)], Fb)
        for cp in pend:
            cp.wait()
        if c + 1 < N_CHUNKS:
            pend = start_gathers(c + 1, 1 - b)
        Ab, Bb, Fbb, EEb = A[b], B[b], Fb, Fb

        def edge_step(r, carry):
            sacc = jnp.zeros((16,), F32)
            for k in range(HID // 16):
                x = Ab[r, pl.ds(k * 16, 16)] + Bb[r, pl.ds(k * 16, 16)] \
                    + Fbb[r, pl.ds(k * 16, 16)]
                eev = jnp.maximum(x, x * 0.01)
                EEb[r, pl.ds(k * 16, 16)] = eev
                sacc = sacc + eev * attn_v[pl.ds(k * 16, 16)]
            for ix in xor_idx:
                sacc = sacc + _lane_perm(sacc, ix)
            exv = jnp.exp(sacc)
            for k in range(HID // 16):
                R[r, pl.ds(k * 16, 16)] = Ab[r, pl.ds(HID + k * 16, 16)] * exv
            R[r, pl.ds(HID, 16)] = jnp.where(lane == 0, exv, 0.0)
            return carry

        lax.fori_loop(0, CHUNK, edge_step, 0)
        pltpu.sync_copy(R, acc_sh.at[dbuf[b]], add=True)
        base = wid * EPT + c * CHUNK
        pltpu.sync_copy(EEb, ee_out.at[pl.ds(base, CHUNK)])
    plsc.subcore_barrier()
    pltpu.sync_copy(acc_sh.at[pl.ds(sid * CHUNK, CHUNK)],
                    acc_out.at[cid, pl.ds(sid * CHUNK, CHUNK)])


def _vmem2(shape, dtype):
    return (pltpu.VMEM(shape, dtype), pltpu.VMEM(shape, dtype))


@functools.cache
def _edge_call():
    return pl.kernel(
        _edge_body,
        out_type=(
            jax.ShapeDtypeStruct((N_EDGES, HID), F32),
            jax.ShapeDtypeStruct((2, N_NODES, ACC_W), F32),
        ),
        mesh=_sc_mesh(),
        scratch_types=[
            pltpu.VMEM((HID,), F32),
            _vmem2((CHUNK,), jnp.int32),
            _vmem2((CHUNK,), jnp.int32),
            _vmem2((CHUNK, ACC_W), F32),
            _vmem2((CHUNK, ACC_W), F32),
            pltpu.VMEM((CHUNK, HID), F32),
            pltpu.VMEM((CHUNK, ACC_W), F32),
            pltpu.VMEM_SHARED((N_NODES, ACC_W), F32),
            (pltpu.SemaphoreType.DMA, pltpu.SemaphoreType.DMA),
        ],
    )


def _dotT(x, w):
    return lax.dot_general(x, w, (((1,), (1,)), ((), ())),
                           preferred_element_type=F32)


def _node_prep0_body(h0_ref, wni_ref, wnj_ref, wns_ref,
                     h_ref, fsrc_ref, fdst_ref):
    h = jnp.maximum(h0_ref[:, :HID], 0.0)
    h_ref[...] = h
    fni = _dotT(h, wni_ref[...])
    fnj = _dotT(h, wnj_ref[...])
    ft = _dotT(h, wns_ref[...])
    fsrc_ref[...] = jnp.concatenate([fni, ft], axis=1)
    fdst_ref[...] = jnp.concatenate([fnj, jnp.zeros_like(fnj)], axis=1)


def _node_prep_body(acc0_ref, acc1_ref, wni_ref, wnj_ref, wns_ref,
                    h_ref, fsrc_ref, fdst_ref):
    acc = acc0_ref[...] + acc1_ref[...]
    num = acc[:, :HID]
    den = acc[:, HID:HID + 1]
    h = jnp.maximum(num / (den + 1e-16), 0.0)
    h_ref[...] = h
    fni = _dotT(h, wni_ref[...])
    fnj = _dotT(h, wnj_ref[...])
    ft = _dotT(h, wns_ref[...])
    fsrc_ref[...] = jnp.concatenate([fni, ft], axis=1)
    fdst_ref[...] = jnp.concatenate([fnj, jnp.zeros_like(fnj)], axis=1)


def _finalize_body(acc0_ref, acc1_ref, h_ref):
    acc = acc0_ref[...] + acc1_ref[...]
    h_ref[...] = jnp.maximum(acc[:, :HID] / (acc[:, HID:HID + 1] + 1e-16), 0.0)


def _edge_mm_body(e_ref, w_ref, out_ref):
    out_ref[...] = _dotT(e_ref[:, :HID], w_ref[...])


def _attn_proj_body(x_ref, wqk_ref, wv_ref, qk_ref, kn_ref, v_ref):
    x = x_ref[...]
    qk = _dotT(x, wqk_ref[...])
    qk_ref[...] = qk
    v_ref[...] = _dotT(x, wv_ref[...])
    for hd in range(HEADS):
        s = qk[:, hd * DH:(hd + 1) * DH]
        nrm = jnp.sqrt(jnp.sum(s * s, axis=1, keepdims=True))
        kn_ref[:, hd * DH:(hd + 1) * DH] = s / jnp.maximum(nrm, 1e-12)


def _attention_body(qk_ref, kn_ref, v_ref, out_ref):
    qb = pl.program_id(1)
    q = qk_ref[...]
    scores = lax.dot_general(q, kn_ref[...], (((1,), (1,)), ((), ())),
                             preferred_element_type=F32) * (DH ** -0.5)
    row = lax.broadcasted_iota(jnp.int32, scores.shape, 0) + qb * scores.shape[0]
    col = lax.broadcasted_iota(jnp.int32, scores.shape, 1)
    scores = jnp.where(col == row, -5e4, scores)
    scores = jnp.where(col <= row, scores, NEG_BIG)
    m = jnp.max(scores, axis=1, keepdims=True)
    p = jnp.exp(scores - m)
    out = jnp.dot(p, v_ref[...], preferred_element_type=F32)
    out_ref[...] = out / jnp.sum(p, axis=1, keepdims=True)


def _erf(x):
    a1, a2, a3 = 0.254829592, -0.284496736, 1.421413741
    a4, a5, p = -1.453152027, 1.061405429, 0.3275911
    s = jnp.sign(x)
    ax = jnp.abs(x)
    t = 1.0 / (1.0 + p * ax)
    y = 1.0 - (((((a5 * t + a4) * t) + a3) * t + a2) * t + a1) * t * jnp.exp(-ax * ax)
    return s * y


def _gelu(x):
    return 0.5 * x * (1.0 + _erf(x * (2.0 ** -0.5)))


def _post_attn_body(x_ref, wout_ref, bout_ref, wli1_ref, bli1_ref,
                    wli2_ref, bli2_ref, wlf1_ref, blf1_ref, out_ref):
    a = _dotT(x_ref[...], wout_ref[...]) + bout_ref[...]
    a = _gelu(_dotT(a, wli1_ref[...]) + bli1_ref[...])
    a = _gelu(_dotT(a, wli2_ref[...]) + bli2_ref[...])
    out_ref[...] = jnp.maximum(_dotT(a, wlf1_ref[...]) + blf1_ref[...], 0.0)


def _sortpool_body(h_ref, out_ref):
    feat = h_ref[...]
    lane = lax.broadcasted_iota(jnp.int32, (N_NODES, HID), 1)
    r64 = lax.broadcasted_iota(jnp.int32, (HID, HID), 0)
    c64 = lax.broadcasted_iota(jnp.int32, (HID, HID), 1)
    k = 2
    while k <= HID:
        j = k // 2
        while j >= 1:
            pj = (jnp.bitwise_xor(r64, j) == c64).astype(F32)
            xp = jnp.dot(feat, pj, preferred_element_type=F32)
            bitj0 = (lane & j) == 0
            dirup = (lane & k) == 0
            cond = bitj0 == dirup
            feat = jnp.where(cond, jnp.minimum(feat, xp),
                             jnp.maximum(feat, xp))
            j //= 2
        k *= 2
    kcol = feat[:, HID - 1:HID]
    sub = lax.broadcasted_iota(jnp.int32, (N_NODES, NPG), 0)
    ln16 = lax.broadcasted_iota(jnp.int32, (N_NODES, NPG), 1)
    m16 = jnp.where((sub % NPG) == ln16, kcol, 0.0)
    gsel_r = lax.broadcasted_iota(jnp.int32, (N_GRAPHS, N_NODES), 0)
    gsel_c = lax.broadcasted_iota(jnp.int32, (N_GRAPHS, N_NODES), 1)
    gsel = ((gsel_c // NPG) == gsel_r).astype(F32)
    kk = jnp.dot(gsel, m16, preferred_element_type=F32)
    l16 = lax.broadcasted_iota(jnp.int32, (N_GRAPHS, NPG), 1)
    rank = jnp.zeros((N_GRAPHS, NPG), jnp.int32)
    for j in range(NPG):
        kj = kk[:, j:j + 1]
        gt = (kj > kk).astype(jnp.int32)
        eq = jnp.logical_and(kj == kk, j < l16).astype(jnp.int32)
        rank = rank + gt + eq
    q_r = lax.broadcasted_iota(jnp.int32, (NPG, N_NODES), 0)
    q_c = lax.broadcasted_iota(jnp.int32, (NPG, N_NODES), 1)
    qsel = ((q_c % NPG) == q_r).astype(F32)
    rexp = jnp.dot(rank.astype(F32), qsel,
                   preferred_element_type=F32)
    gmask = ((gsel_c // NPG) == gsel_r).astype(F32)
    ones_row = jnp.ones((1, N_GRAPHS), F32)
    rank_row = jnp.dot(ones_row, rexp * gmask,
                       preferred_element_type=F32)
    blk = 256
    for b in range(N_NODES // blk):
        pr = lax.broadcasted_iota(jnp.int32, (blk, N_NODES), 0) + b * blk
        pc = lax.broadcasted_iota(jnp.int32, (blk, N_NODES), 1)
        same_g = (pr // NPG) == (pc // NPG)
        sel = jnp.logical_and(same_g, rank_row == (pr % NPG).astype(F32))
        out_ref[pl.ds(b * blk, blk), :] = jnp.dot(
            sel.astype(F32), feat, preferred_element_type=F32)


def _gat_head_body(hsp_ref, hspT_ref, srow_ref, drow_ref, scol_ref, dcol_ref,
                   wg_ref, gal_ref, galr_ref, gar_ref, garr_ref, gb_ref,
                   wlin_ref, blin_ref, wlin1_ref, blin1_ref,
                   wcls_ref, bcls_ref, out_ref):
    hsp = hsp_ref[...]
    hspT = hspT_ref[...]
    wg = wg_ref[...]
    ft = _dotT(hsp, wg)
    ftT = jnp.dot(wg, hspT, preferred_element_type=F32)
    el_c = jnp.dot(ft, gal_ref[...], preferred_element_type=F32)
    er_c = jnp.dot(ft, gar_ref[...], preferred_element_type=F32)
    el_r = jnp.dot(galr_ref[...], ftT, preferred_element_type=F32)
    er_r = jnp.dot(garr_ref[...], ftT, preferred_element_type=F32)
    srow = srow_ref[...]
    drow = drow_ref[...]
    scol = scol_ref[...]
    dcol = dcol_ref[...]
    n_r = lax.broadcasted_iota(jnp.int32, (N_GRAPHS, 8 * N_GRAPHS), 0)
    n_c = lax.broadcasted_iota(jnp.int32, (8 * N_GRAPHS, N_GRAPHS), 1)
    o_src = (srow == n_r).astype(F32)
    o_dst = (drow == n_r).astype(F32)
    o_srcT = (scol == n_c).astype(F32)
    o_dstT = (dcol == n_c).astype(F32)
    sc_r = jnp.dot(el_r, o_src, preferred_element_type=F32) \
        + jnp.dot(er_r, o_dst, preferred_element_type=F32)
    sc_r = jnp.maximum(sc_r, sc_r * 0.2)
    sc_c = jnp.dot(o_srcT, el_c, preferred_element_type=F32) \
        + jnp.dot(o_dstT, er_c, preferred_element_type=F32)
    sc_c = jnp.maximum(sc_c, sc_c * 0.2)
    mmat = jnp.where(o_dst > 0.5, sc_r, NEG_BIG)
    m_c = jnp.max(mmat, axis=1, keepdims=True)
    mmatT = jnp.where(o_dstT > 0.5, sc_c, NEG_BIG)
    m_r = jnp.max(mmatT, axis=0, keepdims=True)
    m_c = jnp.where(m_c <= NEG_BIG, 0.0, m_c)
    m_r = jnp.where(m_r <= NEG_BIG, 0.0, m_r)
    ex_c = jnp.exp(sc_c - jnp.dot(o_dstT, m_c, preferred_element_type=F32))
    s_c = jnp.dot(o_dst, ex_c, preferred_element_type=F32)
    den_c = jnp.dot(o_dstT, s_c, preferred_element_type=F32) + 1e-16
    a_c = ex_c / den_c
    ft_src = jnp.dot(o_srcT, ft, preferred_element_type=F32)
    out = jnp.dot(o_dst, ft_src * a_c, preferred_element_type=F32)
    h = jnp.maximum(out + gb_ref[...], 0.0)
    h = jnp.maximum(_dotT(h, wlin_ref[...]) + blin_ref[...], 0.0)
    h = jnp.maximum(_dotT(h, wlin1_ref[...]) + blin1_ref[...], 0.0)
    out_ref[...] = _dotT(h, wcls_ref[...]) + bcls_ref[...]


def _tc_call(body, out_shape, **kw):
    return pl.pallas_call(body, out_shape=out_shape, **kw)


def kernel(token_emb, e_token_emb, egat_Wns, egat_Wni, egat_Wfij, egat_Wnj,
           egat_attn, egat_bias, Wqk, Wv, Wout, bout, Wli1, bli1, Wli2, bli2,
           Wlf1, blf1, Wg, gal, gar, gbias, Wlin, blin, Wlin1, blin1,
           Wcls, bcls, h_tok, e_tok, edge_index, fg_edge_index):
    f32 = jnp.float32
    src = edge_index[0]
    dst = edge_index[1]
    zeros_init = jnp.zeros((CHUNK, ACC_W), f32)
    htab = jnp.pad(token_emb.astype(f32), ((0, 0), (0, ACC_W - HID)))
    etab = jnp.pad(e_token_emb.astype(f32), ((0, 0), (0, ACC_W - HID)))

    h0_raw, e = _emb_call()(h_tok, e_tok, htab, etab)

    node_shapes = (
        jax.ShapeDtypeStruct((N_NODES, HID), f32),
        jax.ShapeDtypeStruct((N_NODES, 2 * HID), f32),
        jax.ShapeDtypeStruct((N_NODES, 2 * HID), f32),
    )

    hs_list = []
    acc = None
    for l in range(N_LAYERS):
        if l == 0:
            h, fsrc, fdst = _tc_call(_node_prep0_body, node_shapes)(
                h0_raw, egat_Wni[l], egat_Wnj[l], egat_Wns[l])
        else:
            h, fsrc, fdst = _tc_call(_node_prep_body, node_shapes)(
                acc[0], acc[1], egat_Wni[l], egat_Wnj[l], egat_Wns[l])
            hs_list.append(h)
        e_w = e.shape[1]
        ffij = _tc_call(
            _edge_mm_body,
            jax.ShapeDtypeStruct((N_EDGES, HID), f32),
            grid=(8,),
            in_specs=[
                pl.BlockSpec((N_EDGES // 8, e_w), lambda i: (i, 0)),
                pl.BlockSpec((HID, HID), lambda i: (0, 0)),
            ],
            out_specs=pl.BlockSpec((N_EDGES // 8, HID), lambda i: (i, 0)),
        )(e, egat_Wfij[l])
        e, acc = _edge_call()(fsrc, fdst, ffij, src, dst,
                              egat_attn[l], zeros_init)

    h8 = _tc_call(_finalize_body, jax.ShapeDtypeStruct((N_NODES, HID), f32))(
        acc[0], acc[1])
    hs_list.append(h8)
    hs = jnp.concatenate(hs_list, axis=-1)

    qk, kn, v = _tc_call(
        _attn_proj_body,
        (jax.ShapeDtypeStruct((N_NODES, ATT_DIM), f32),) * 3,
        grid=(8,),
        in_specs=[
            pl.BlockSpec((N_NODES // 8, ATT_DIM), lambda i: (i, 0)),
            pl.BlockSpec((ATT_DIM, ATT_DIM), lambda i: (0, 0)),
            pl.BlockSpec((ATT_DIM, ATT_DIM), lambda i: (0, 0)),
        ],
        out_specs=[pl.BlockSpec((N_NODES // 8, ATT_DIM), lambda i: (i, 0))] * 3,
    )(hs, Wqk, Wv)

    att = _tc_call(
        _attention_body,
        jax.ShapeDtypeStruct((N_NODES, ATT_DIM), f32),
        grid=(HEADS, 8),
        in_specs=[
            pl.BlockSpec((N_NODES // 8, DH), lambda hd, q: (q, hd)),
            pl.BlockSpec((N_NODES, DH), lambda hd, q: (0, hd)),
            pl.BlockSpec((N_NODES, DH), lambda hd, q: (0, hd)),
        ],
        out_specs=pl.BlockSpec((N_NODES // 8, DH), lambda hd, q: (q, hd)),
    )(qk, kn, v)

    hp = _tc_call(
        _post_attn_body,
        jax.ShapeDtypeStruct((N_NODES, HID), f32),
        grid=(8,),
        in_specs=[
            pl.BlockSpec((N_NODES // 8, ATT_DIM), lambda i: (i, 0)),
            pl.BlockSpec((ATT_DIM, ATT_DIM), lambda i: (0, 0)),
            pl.BlockSpec((1, ATT_DIM), lambda i: (0, 0)),
            pl.BlockSpec((HID, ATT_DIM), lambda i: (0, 0)),
            pl.BlockSpec((1, HID), lambda i: (0, 0)),
            pl.BlockSpec((ATT_DIM, HID), lambda i: (0, 0)),
            pl.BlockSpec((1, ATT_DIM), lambda i: (0, 0)),
            pl.BlockSpec((HID, ATT_DIM), lambda i: (0, 0)),
            pl.BlockSpec((1, HID), lambda i: (0, 0)),
        ],
        out_specs=pl.BlockSpec((N_NODES // 8, HID), lambda i: (i, 0)),
    )(att, Wout, bout[None], Wli1, bli1[None], Wli2, bli2[None],
      Wlf1, blf1[None])

    sp = _tc_call(_sortpool_body,
                  jax.ShapeDtypeStruct((N_NODES, HID), f32))(hp)
    hsp = sp.reshape(N_GRAPHS, NPG * HID)

    logits = _tc_call(_gat_head_body,
                      jax.ShapeDtypeStruct((N_GRAPHS, 2), f32))(
        hsp, hsp.T,
        fg_edge_index[0:1, :], fg_edge_index[1:2, :],
        fg_edge_index.T[:, 0:1], fg_edge_index.T[:, 1:2],
        Wg, gal[:, None], gal[None, :], gar[:, None], gar[None, :],
        gbias[None, :],
        Wlin, blin[None, :], Wlin1, blin1[None, :], Wcls, bcls[None, :])

    return logits.reshape(-1, 2)


# speedup vs baseline: 14.2700x; 1.3259x over previous
import functools

import jax
import jax.numpy as jnp
from jax import lax
from jax.experimental import pallas as pl
from jax.experimental.pallas import tpu as pltpu
from jax.experimental.pallas import tpu_sc as plsc

N_NODES = 2048
N_EDGES = 65536
N_GRAPHS = 128
NPG = 16
HID = 64
N_LAYERS = 8
HEADS = 4
ATT_DIM = N_LAYERS * HID
DH = ATT_DIM // HEADS

F32 = jnp.float32
NEG_BIG = float(jnp.finfo(jnp.float32).min)

N_TILES = 32
EPT = N_EDGES // N_TILES
CHUNK = 128
N_CHUNKS = EPT // CHUNK
ACC_W = 128

@functools.cache
def _sc_mesh():
    return plsc.VectorSubcoreMesh(core_axis_name="c", subcore_axis_name="s")


def _lane_perm(v, idx):
    return lax.gather(
        v, idx[:, None],
        lax.GatherDimensionNumbers(offset_dims=(), collapsed_slice_dims=(0,),
                                   start_index_map=(0,)),
        slice_sizes=(1,), mode=lax.GatherScatterMode.PROMISE_IN_BOUNDS)


def _emb_body(htok, etok, htab, etab, h0_out, e0_out,
              hib, hrows, ib, rows, sem):
    cid = lax.axis_index("c")
    sid = lax.axis_index("s")
    wid = sid * 2 + cid
    hbase = wid * (N_NODES // N_TILES)
    pltpu.sync_copy(htok.at[pl.ds(hbase, N_NODES // N_TILES)], hib)
    pltpu.async_copy(htab.at[hib], hrows, sem).wait()
    pltpu.sync_copy(hrows, h0_out.at[pl.ds(hbase, N_NODES // N_TILES)])
    for c in range(N_CHUNKS):
        base = wid * EPT + c * CHUNK
        pltpu.sync_copy(etok.at[pl.ds(base, CHUNK)], ib)
        pltpu.async_copy(etab.at[ib], rows, sem).wait()
        pltpu.sync_copy(rows, e0_out.at[pl.ds(base, CHUNK)])


@functools.cache
def _emb_call():
    return pl.kernel(
        _emb_body,
        out_type=(
            jax.ShapeDtypeStruct((N_NODES, ACC_W), F32),
            jax.ShapeDtypeStruct((N_EDGES, ACC_W), F32),
        ),
        mesh=_sc_mesh(),
        scratch_types=[
            pltpu.VMEM((N_NODES // N_TILES,), jnp.int32),
            pltpu.VMEM((N_NODES // N_TILES, ACC_W), F32),
            pltpu.VMEM((CHUNK,), jnp.int32),
            pltpu.VMEM((CHUNK, ACC_W), F32),
            pltpu.SemaphoreType.DMA,
        ],
    )


def _edge_body(fsrc, fdst, ffij, src, dst, attn, zeros, ee_out, acc_out,
               attn_v, sbuf, dbuf, A, B, Fb, R, acc_sh, gsem):
    cid = lax.axis_index("c")
    sid = lax.axis_index("s")
    wid = sid * 2 + cid
    pltpu.sync_copy(zeros, acc_sh.at[pl.ds(sid * CHUNK, CHUNK)])
    pltpu.sync_copy(attn, attn_v)

    def zrow(r, carry):
        for k in range(HID // 16, ACC_W // 16):
            R[r, pl.ds(k * 16, 16)] = jnp.zeros((16,), F32)
        return carry

    lax.fori_loop(0, CHUNK, zrow, 0)
    plsc.subcore_barrier()
    lane = lax.broadcasted_iota(jnp.int32, (16,), 0)
    xor_idx = [jnp.bitwise_xor(lane, d) for d in (1, 2, 4, 8)]

    def start_gathers(c, b):
        base = wid * EPT + c * CHUNK
        pltpu.sync_copy(src.at[pl.ds(base, CHUNK)], sbuf[b])
        pltpu.sync_copy(dst.at[pl.ds(base, CHUNK)], dbuf[b])
        return (pltpu.async_copy(fsrc.at[sbuf[b]], A[b], gsem[b]),
                pltpu.async_copy(fdst.at[dbuf[b]], B[b], gsem[b]))

    pend = start_gathers(0, 0)
    for c in range(N_CHUNKS):
        b = c % 2
        base_c = wid * EPT + c * CHUNK
        pltpu.sync_copy(ffij.at[pl.ds(base_c, CHUNK)], Fb)
        for cp in pend:
            cp.wait()
        if c + 1 < N_CHUNKS:
            pend = start_gathers(c + 1, 1 - b)
        Ab, Bb, Fbb, EEb = A[b], B[b], Fb, Fb
        attn_regs = [attn_v[pl.ds(k * 16, 16)] for k in range(HID // 16)]

        @functools.partial(plsc.parallel_loop, 0, CHUNK, unroll=4)
        def edge_step(r):
            sacc = jnp.zeros((16,), F32)
            for k in range(HID // 16):
                x = Ab[r, pl.ds(k * 16, 16)] + Bb[r, pl.ds(k * 16, 16)] \
                    + Fbb[r, pl.ds(k * 16, 16)]
                eev = jnp.maximum(x, x * 0.01)
                EEb[r, pl.ds(k * 16, 16)] = eev
                sacc = sacc + eev * attn_regs[k]
            for ix in xor_idx:
                sacc = sacc + _lane_perm(sacc, ix)
            exv = jnp.exp(sacc)
            for k in range(HID // 16):
                R[r, pl.ds(k * 16, 16)] = Ab[r, pl.ds(HID + k * 16, 16)] * exv
            R[r, pl.ds(HID, 16)] = jnp.where(lane == 0, exv, 0.0)
        pltpu.sync_copy(R, acc_sh.at[dbuf[b]], add=True)
        base = wid * EPT + c * CHUNK
        pltpu.sync_copy(EEb, ee_out.at[pl.ds(base, CHUNK)])
    plsc.subcore_barrier()
    pltpu.sync_copy(acc_sh.at[pl.ds(sid * CHUNK, CHUNK)],
                    acc_out.at[cid, pl.ds(sid * CHUNK, CHUNK)])


def _vmem2(shape, dtype):
    return (pltpu.VMEM(shape, dtype), pltpu.VMEM(shape, dtype))


@functools.cache
def _edge_call():
    return pl.kernel(
        _edge_body,
        out_type=(
            jax.ShapeDtypeStruct((N_EDGES, HID), F32),
            jax.ShapeDtypeStruct((2, N_NODES, ACC_W), F32),
        ),
        mesh=_sc_mesh(),
        scratch_types=[
            pltpu.VMEM((HID,), F32),
            _vmem2((CHUNK,), jnp.int32),
            _vmem2((CHUNK,), jnp.int32),
            _vmem2((CHUNK, ACC_W), F32),
            _vmem2((CHUNK, ACC_W), F32),
            pltpu.VMEM((CHUNK, HID), F32),
            pltpu.VMEM((CHUNK, ACC_W), F32),
            pltpu.VMEM_SHARED((N_NODES, ACC_W), F32),
            (pltpu.SemaphoreType.DMA, pltpu.SemaphoreType.DMA),
        ],
    )


def _dotT(x, w):
    return lax.dot_general(x, w, (((1,), (1,)), ((), ())),
                           preferred_element_type=F32)


def _node_prep0_body(h0_ref, wni_ref, wnj_ref, wns_ref,
                     h_ref, fsrc_ref, fdst_ref):
    h = jnp.maximum(h0_ref[:, :HID], 0.0)
    h_ref[...] = h
    fni = _dotT(h, wni_ref[...])
    fnj = _dotT(h, wnj_ref[...])
    ft = _dotT(h, wns_ref[...])
    fsrc_ref[...] = jnp.concatenate([fni, ft], axis=1)
    fdst_ref[...] = jnp.concatenate([fnj, jnp.zeros_like(fnj)], axis=1)


def _node_prep_body(acc0_ref, acc1_ref, wni_ref, wnj_ref, wns_ref,
                    h_ref, fsrc_ref, fdst_ref):
    acc = acc0_ref[...] + acc1_ref[...]
    num = acc[:, :HID]
    den = acc[:, HID:HID + 1]
    h = jnp.maximum(num / (den + 1e-16), 0.0)
    h_ref[...] = h
    fni = _dotT(h, wni_ref[...])
    fnj = _dotT(h, wnj_ref[...])
    ft = _dotT(h, wns_ref[...])
    fsrc_ref[...] = jnp.concatenate([fni, ft], axis=1)
    fdst_ref[...] = jnp.concatenate([fnj, jnp.zeros_like(fnj)], axis=1)


def _finalize_body(acc0_ref, acc1_ref, h_ref):
    acc = acc0_ref[...] + acc1_ref[...]
    h_ref[...] = jnp.maximum(acc[:, :HID] / (acc[:, HID:HID + 1] + 1e-16), 0.0)


def _edge_mm_body(e_ref, w_ref, out_ref):
    out_ref[...] = _dotT(e_ref[:, :HID], w_ref[...])


def _attn_proj_body(x_ref, wqk_ref, wv_ref, qk_ref, kn_ref, v_ref):
    x = x_ref[...]
    qk = _dotT(x, wqk_ref[...])
    qk_ref[...] = qk
    v_ref[...] = _dotT(x, wv_ref[...])
    for hd in range(HEADS):
        s = qk[:, hd * DH:(hd + 1) * DH]
        nrm = jnp.sqrt(jnp.sum(s * s, axis=1, keepdims=True))
        kn_ref[:, hd * DH:(hd + 1) * DH] = s / jnp.maximum(nrm, 1e-12)


def _attention_body(qk_ref, kn_ref, v_ref, out_ref):
    qb = pl.program_id(1)
    q = qk_ref[...]
    scores = lax.dot_general(q, kn_ref[...], (((1,), (1,)), ((), ())),
                             preferred_element_type=F32) * (DH ** -0.5)
    row = lax.broadcasted_iota(jnp.int32, scores.shape, 0) + qb * scores.shape[0]
    col = lax.broadcasted_iota(jnp.int32, scores.shape, 1)
    scores = jnp.where(col == row, -5e4, scores)
    scores = jnp.where(col <= row, scores, NEG_BIG)
    m = jnp.max(scores, axis=1, keepdims=True)
    p = jnp.exp(scores - m)
    out = jnp.dot(p, v_ref[...], preferred_element_type=F32)
    out_ref[...] = out / jnp.sum(p, axis=1, keepdims=True)


def _erf(x):
    a1, a2, a3 = 0.254829592, -0.284496736, 1.421413741
    a4, a5, p = -1.453152027, 1.061405429, 0.3275911
    s = jnp.sign(x)
    ax = jnp.abs(x)
    t = 1.0 / (1.0 + p * ax)
    y = 1.0 - (((((a5 * t + a4) * t) + a3) * t + a2) * t + a1) * t * jnp.exp(-ax * ax)
    return s * y


def _gelu(x):
    return 0.5 * x * (1.0 + _erf(x * (2.0 ** -0.5)))


def _post_attn_body(x_ref, wout_ref, bout_ref, wli1_ref, bli1_ref,
                    wli2_ref, bli2_ref, wlf1_ref, blf1_ref, out_ref):
    a = _dotT(x_ref[...], wout_ref[...]) + bout_ref[...]
    a = _gelu(_dotT(a, wli1_ref[...]) + bli1_ref[...])
    a = _gelu(_dotT(a, wli2_ref[...]) + bli2_ref[...])
    out_ref[...] = jnp.maximum(_dotT(a, wlf1_ref[...]) + blf1_ref[...], 0.0)


def _sortpool_body(h_ref, out_ref):
    feat = h_ref[...]
    lane = lax.broadcasted_iota(jnp.int32, (N_NODES, HID), 1)
    r64 = lax.broadcasted_iota(jnp.int32, (HID, HID), 0)
    c64 = lax.broadcasted_iota(jnp.int32, (HID, HID), 1)
    k = 2
    while k <= HID:
        j = k // 2
        while j >= 1:
            pj = (jnp.bitwise_xor(r64, j) == c64).astype(F32)
            xp = jnp.dot(feat, pj, preferred_element_type=F32)
            bitj0 = (lane & j) == 0
            dirup = (lane & k) == 0
            cond = bitj0 == dirup
            feat = jnp.where(cond, jnp.minimum(feat, xp),
                             jnp.maximum(feat, xp))
            j //= 2
        k *= 2
    kcol = feat[:, HID - 1:HID]
    sub = lax.broadcasted_iota(jnp.int32, (N_NODES, NPG), 0)
    ln16 = lax.broadcasted_iota(jnp.int32, (N_NODES, NPG), 1)
    m16 = jnp.where((sub % NPG) == ln16, kcol, 0.0)
    gsel_r = lax.broadcasted_iota(jnp.int32, (N_GRAPHS, N_NODES), 0)
    gsel_c = lax.broadcasted_iota(jnp.int32, (N_GRAPHS, N_NODES), 1)
    gsel = ((gsel_c // NPG) == gsel_r).astype(F32)
    kk = jnp.dot(gsel, m16, preferred_element_type=F32)
    l16 = lax.broadcasted_iota(jnp.int32, (N_GRAPHS, NPG), 1)
    rank = jnp.zeros((N_GRAPHS, NPG), jnp.int32)
    for j in range(NPG):
        kj = kk[:, j:j + 1]
        gt = (kj > kk).astype(jnp.int32)
        eq = jnp.logical_and(kj == kk, j < l16).astype(jnp.int32)
        rank = rank + gt + eq
    q_r = lax.broadcasted_iota(jnp.int32, (NPG, N_NODES), 0)
    q_c = lax.broadcasted_iota(jnp.int32, (NPG, N_NODES), 1)
    qsel = ((q_c % NPG) == q_r).astype(F32)
    rexp = jnp.dot(rank.astype(F32), qsel,
                   preferred_element_type=F32)
    gmask = ((gsel_c // NPG) == gsel_r).astype(F32)
    ones_row = jnp.ones((1, N_GRAPHS), F32)
    rank_row = jnp.dot(ones_row, rexp * gmask,
                       preferred_element_type=F32)
    blk = 256
    for b in range(N_NODES // blk):
        pr = lax.broadcasted_iota(jnp.int32, (blk, N_NODES), 0) + b * blk
        pc = lax.broadcasted_iota(jnp.int32, (blk, N_NODES), 1)
        same_g = (pr // NPG) == (pc // NPG)
        sel = jnp.logical_and(same_g, rank_row == (pr % NPG).astype(F32))
        out_ref[pl.ds(b * blk, blk), :] = jnp.dot(
            sel.astype(F32), feat, preferred_element_type=F32)


def _gat_head_body(hsp_ref, hspT_ref, srow_ref, drow_ref, scol_ref, dcol_ref,
                   wg_ref, gal_ref, galr_ref, gar_ref, garr_ref, gb_ref,
                   wlin_ref, blin_ref, wlin1_ref, blin1_ref,
                   wcls_ref, bcls_ref, out_ref):
    hsp = hsp_ref[...]
    hspT = hspT_ref[...]
    wg = wg_ref[...]
    ft = _dotT(hsp, wg)
    ftT = jnp.dot(wg, hspT, preferred_element_type=F32)
    el_c = jnp.dot(ft, gal_ref[...], preferred_element_type=F32)
    er_c = jnp.dot(ft, gar_ref[...], preferred_element_type=F32)
    el_r = jnp.dot(galr_ref[...], ftT, preferred_element_type=F32)
    er_r = jnp.dot(garr_ref[...], ftT, preferred_element_type=F32)
    srow = srow_ref[...]
    drow = drow_ref[...]
    scol = scol_ref[...]
    dcol = dcol_ref[...]
    n_r = lax.broadcasted_iota(jnp.int32, (N_GRAPHS, 8 * N_GRAPHS), 0)
    n_c = lax.broadcasted_iota(jnp.int32, (8 * N_GRAPHS, N_GRAPHS), 1)
    o_src = (srow == n_r).astype(F32)
    o_dst = (drow == n_r).astype(F32)
    o_srcT = (scol == n_c).astype(F32)
    o_dstT = (dcol == n_c).astype(F32)
    sc_r = jnp.dot(el_r, o_src, preferred_element_type=F32) \
        + jnp.dot(er_r, o_dst, preferred_element_type=F32)
    sc_r = jnp.maximum(sc_r, sc_r * 0.2)
    sc_c = jnp.dot(o_srcT, el_c, preferred_element_type=F32) \
        + jnp.dot(o_dstT, er_c, preferred_element_type=F32)
    sc_c = jnp.maximum(sc_c, sc_c * 0.2)
    mmat = jnp.where(o_dst > 0.5, sc_r, NEG_BIG)
    m_c = jnp.max(mmat, axis=1, keepdims=True)
    mmatT = jnp.where(o_dstT > 0.5, sc_c, NEG_BIG)
    m_r = jnp.max(mmatT, axis=0, keepdims=True)
    m_c = jnp.where(m_c <= NEG_BIG, 0.0, m_c)
    m_r = jnp.where(m_r <= NEG_BIG, 0.0, m_r)
    ex_c = jnp.exp(sc_c - jnp.dot(o_dstT, m_c, preferred_element_type=F32))
    s_c = jnp.dot(o_dst, ex_c, preferred_element_type=F32)
    den_c = jnp.dot(o_dstT, s_c, preferred_element_type=F32) + 1e-16
    a_c = ex_c / den_c
    ft_src = jnp.dot(o_srcT, ft, preferred_element_type=F32)
    out = jnp.dot(o_dst, ft_src * a_c, preferred_element_type=F32)
    h = jnp.maximum(out + gb_ref[...], 0.0)
    h = jnp.maximum(_dotT(h, wlin_ref[...]) + blin_ref[...], 0.0)
    h = jnp.maximum(_dotT(h, wlin1_ref[...]) + blin1_ref[...], 0.0)
    out_ref[...] = _dotT(h, wcls_ref[...]) + bcls_ref[...]


def _tc_call(body, out_shape, **kw):
    return pl.pallas_call(body, out_shape=out_shape, **kw)


def kernel(token_emb, e_token_emb, egat_Wns, egat_Wni, egat_Wfij, egat_Wnj,
           egat_attn, egat_bias, Wqk, Wv, Wout, bout, Wli1, bli1, Wli2, bli2,
           Wlf1, blf1, Wg, gal, gar, gbias, Wlin, blin, Wlin1, blin1,
           Wcls, bcls, h_tok, e_tok, edge_index, fg_edge_index):
    f32 = jnp.float32
    src = edge_index[0]
    dst = edge_index[1]
    zeros_init = jnp.zeros((CHUNK, ACC_W), f32)
    htab = jnp.pad(token_emb.astype(f32), ((0, 0), (0, ACC_W - HID)))
    etab = jnp.pad(e_token_emb.astype(f32), ((0, 0), (0, ACC_W - HID)))

    h0_raw, e = _emb_call()(h_tok, e_tok, htab, etab)

    node_shapes = (
        jax.ShapeDtypeStruct((N_NODES, HID), f32),
        jax.ShapeDtypeStruct((N_NODES, 2 * HID), f32),
        jax.ShapeDtypeStruct((N_NODES, 2 * HID), f32),
    )

    hs_list = []
    acc = None
    for l in range(N_LAYERS):
        if l == 0:
            h, fsrc, fdst = _tc_call(_node_prep0_body, node_shapes)(
                h0_raw, egat_Wni[l], egat_Wnj[l], egat_Wns[l])
        else:
            h, fsrc, fdst = _tc_call(_node_prep_body, node_shapes)(
                acc[0], acc[1], egat_Wni[l], egat_Wnj[l], egat_Wns[l])
            hs_list.append(h)
        e_w = e.shape[1]
        ffij = _tc_call(
            _edge_mm_body,
            jax.ShapeDtypeStruct((N_EDGES, HID), f32),
            grid=(8,),
            in_specs=[
                pl.BlockSpec((N_EDGES // 8, e_w), lambda i: (i, 0)),
                pl.BlockSpec((HID, HID), lambda i: (0, 0)),
            ],
            out_specs=pl.BlockSpec((N_EDGES // 8, HID), lambda i: (i, 0)),
        )(e, egat_Wfij[l])
        e, acc = _edge_call()(fsrc, fdst, ffij, src, dst,
                              egat_attn[l], zeros_init)

    h8 = _tc_call(_finalize_body, jax.ShapeDtypeStruct((N_NODES, HID), f32))(
        acc[0], acc[1])
    hs_list.append(h8)
    hs = jnp.concatenate(hs_list, axis=-1)

    qk, kn, v = _tc_call(
        _attn_proj_body,
        (jax.ShapeDtypeStruct((N_NODES, ATT_DIM), f32),) * 3,
        grid=(8,),
        in_specs=[
            pl.BlockSpec((N_NODES // 8, ATT_DIM), lambda i: (i, 0)),
            pl.BlockSpec((ATT_DIM, ATT_DIM), lambda i: (0, 0)),
            pl.BlockSpec((ATT_DIM, ATT_DIM), lambda i: (0, 0)),
        ],
        out_specs=[pl.BlockSpec((N_NODES // 8, ATT_DIM), lambda i: (i, 0))] * 3,
    )(hs, Wqk, Wv)

    att = _tc_call(
        _attention_body,
        jax.ShapeDtypeStruct((N_NODES, ATT_DIM), f32),
        grid=(HEADS, 8),
        in_specs=[
            pl.BlockSpec((N_NODES // 8, DH), lambda hd, q: (q, hd)),
            pl.BlockSpec((N_NODES, DH), lambda hd, q: (0, hd)),
            pl.BlockSpec((N_NODES, DH), lambda hd, q: (0, hd)),
        ],
        out_specs=pl.BlockSpec((N_NODES // 8, DH), lambda hd, q: (q, hd)),
    )(qk, kn, v)

    hp = _tc_call(
        _post_attn_body,
        jax.ShapeDtypeStruct((N_NODES, HID), f32),
        grid=(8,),
        in_specs=[
            pl.BlockSpec((N_NODES // 8, ATT_DIM), lambda i: (i, 0)),
            pl.BlockSpec((ATT_DIM, ATT_DIM), lambda i: (0, 0)),
            pl.BlockSpec((1, ATT_DIM), lambda i: (0, 0)),
            pl.BlockSpec((HID, ATT_DIM), lambda i: (0, 0)),
            pl.BlockSpec((1, HID), lambda i: (0, 0)),
            pl.BlockSpec((ATT_DIM, HID), lambda i: (0, 0)),
            pl.BlockSpec((1, ATT_DIM), lambda i: (0, 0)),
            pl.BlockSpec((HID, ATT_DIM), lambda i: (0, 0)),
            pl.BlockSpec((1, HID), lambda i: (0, 0)),
        ],
        out_specs=pl.BlockSpec((N_NODES // 8, HID), lambda i: (i, 0)),
    )(att, Wout, bout[None], Wli1, bli1[None], Wli2, bli2[None],
      Wlf1, blf1[None])

    sp = _tc_call(_sortpool_body,
                  jax.ShapeDtypeStruct((N_NODES, HID), f32))(hp)
    hsp = sp.reshape(N_GRAPHS, NPG * HID)

    logits = _tc_call(_gat_head_body,
                      jax.ShapeDtypeStruct((N_GRAPHS, 2), f32))(
        hsp, hsp.T,
        fg_edge_index[0:1, :], fg_edge_index[1:2, :],
        fg_edge_index.T[:, 0:1], fg_edge_index.T[:, 1:2],
        Wg, gal[:, None], gal[None, :], gar[:, None], gar[None, :],
        gbias[None, :],
        Wlin, blin[None, :], Wlin1, blin1[None, :], Wcls, bcls[None, :])

    return logits.reshape(-1, 2)
